# baseline (speedup 1.0000x reference)
import functools

import jax
import jax.numpy as jnp
from jax import lax
from jax.experimental import pallas as pl
from jax.experimental.pallas import tpu as pltpu
from jax.experimental.pallas import tpu_sc as plsc

N = 10000
E = 160000
D = 256
P = 128
TEMP = 0.1

NC = 2
NS = 16
CH = 128
CPT = 80
EPAD = NS * CH * CPT
RPT = 632
NPAD = NS * RPT


def _deg_body(idx2_hbm, ones_hbm, zero8_hbm, out_hbm, idxv, onesv, hist):
    c = lax.axis_index("c")
    s = lax.axis_index("s")
    pltpu.sync_copy(ones_hbm, onesv)
    pltpu.sync_copy(idx2_hbm.at[c].at[s], idxv)
    pltpu.sync_copy(zero8_hbm, hist.at[pl.ds(s * RPT, RPT)])
    plsc.subcore_barrier()

    def body(j, carry):
        pltpu.sync_copy(onesv, hist.at[idxv.at[j]], add=True)
        return carry

    lax.fori_loop(0, CPT, body, 0)
    plsc.subcore_barrier()
    pltpu.sync_copy(hist.at[pl.ds(s * RPT, RPT)],
                    out_hbm.at[c].at[pl.ds(s * RPT, RPT)])


_HCPT = CPT // 2


def _scatter_body(rows_hbm, src_hbm, dst_hbm, zrow_hbm, out_hbm,
                  gidx_all, didx_all, rows0, rows1, acc,
                  gsem0, gsem1, ssem0, ssem1):
    c = lax.axis_index("c")
    s = lax.axis_index("s")
    rows_bufs = (rows0, rows1)
    gsems = (gsem0, gsem1)
    ssems = (ssem0, ssem1)

    def prep_half(h):
        pltpu.sync_copy(src_hbm.at[s].at[pl.ds(h * _HCPT, _HCPT)], gidx_all)
        pltpu.sync_copy(dst_hbm.at[s].at[pl.ds(h * _HCPT, _HCPT)], didx_all)

        def xform(r, carry):
            for i in range(CH // 16):
                v = gidx_all[r, pl.ds(i * 16, 16)]
                gidx_all[r, pl.ds(i * 16, 16)] = v * 2 + c
            return carry

        lax.fori_loop(0, _HCPT, xform, 0)

    def prime(h):
        pltpu.async_copy(rows_hbm.at[gidx_all.at[0]], rows0, gsems[0])
        pltpu.async_copy(rows_hbm.at[gidx_all.at[1]], rows1, gsems[1])

    def run_half():
        def body(g, carry):
            for b in range(2):
                j = 2 * g + b
                pltpu.make_async_copy(rows_hbm.at[gidx_all.at[0]],
                                      rows_bufs[b], gsems[b]).wait()
                pltpu.sync_copy(rows_bufs[b], acc.at[didx_all.at[j]],
                                add=True)

                @pl.when(j + 2 < _HCPT)
                def _():
                    pltpu.async_copy(rows_hbm.at[gidx_all.at[j + 2]],
                                     rows_bufs[b], gsems[b])
            return carry

        lax.fori_loop(0, _HCPT // 2, body, 0)

    pltpu.sync_copy(zrow_hbm, acc.at[pl.ds(s * RPT, RPT)])
    prep_half(0)
    prime(0)
    plsc.subcore_barrier()
    run_half()
    prep_half(1)
    prime(1)
    run_half()
    plsc.subcore_barrier()
    pltpu.sync_copy(acc.at[pl.ds(s * RPT, RPT)],
                    out_hbm.at[c].at[pl.ds(s * RPT, RPT)])


@functools.cache
def _sc_kernels():
    mesh = plsc.VectorSubcoreMesh(core_axis_name="c", subcore_axis_name="s",
                                  num_cores=NC, num_subcores=NS)
    deg = pl.kernel(
        _deg_body,
        out_type=jax.ShapeDtypeStruct((NC, NPAD, P), jnp.float32),
        mesh=mesh,
        scratch_types=[
            pltpu.VMEM((CPT, CH), jnp.int32),
            pltpu.VMEM((CH, P), jnp.float32),
            pltpu.VMEM_SHARED((NPAD, P), jnp.float32),
        ],
    )
    scat = pl.kernel(
        _scatter_body,
        out_type=jax.ShapeDtypeStruct((NC, NPAD, P), jnp.float32),
        mesh=mesh,
        scratch_types=[
            pltpu.VMEM((_HCPT, CH), jnp.int32),
            pltpu.VMEM((_HCPT, CH), jnp.int32),
            pltpu.VMEM((CH, P), jnp.float32),
            pltpu.VMEM((CH, P), jnp.float32),
            pltpu.VMEM_SHARED((NPAD, P), jnp.float32),
            pltpu.SemaphoreType.DMA,
            pltpu.SemaphoreType.DMA,
            pltpu.SemaphoreType.DMA,
            pltpu.SemaphoreType.DMA,
        ],
    )
    return deg, scat


def _sc_degrees(idx2, ones8, zrow):
    return _sc_kernels()[0](idx2, ones8, zrow)


def _sc_scatter(rows_il, srcp, dstp, zrow):
    return _sc_kernels()[1](rows_il, srcp, dstp, zrow)


def _hn_body(x_ref, dout_ref, hn_ref):
    sc = lax.rsqrt(jnp.maximum(dout_ref[...], 1.0))
    hn_ref[...] = x_ref[...] * sc


def _tc_hn(x, deg_out):
    return pl.pallas_call(
        _hn_body,
        out_shape=jax.ShapeDtypeStruct((N, D), jnp.float32),
    )(x, deg_out)


def _base_body(agg_ref, din_ref, dout_ref, w_ref, b_ref, base_ref, hsn_ref):
    s_in = lax.rsqrt(jnp.maximum(din_ref[...], 1.0))
    scaled = agg_ref[...] * s_in
    base = jnp.dot(scaled, w_ref[...],
                   preferred_element_type=jnp.float32) + b_ref[...]
    base_ref[...] = base
    hsn_ref[...] = base * lax.rsqrt(dout_ref[...] + 1.0)


def _tc_base(agg, deg_in, deg_out, w, b):
    return pl.pallas_call(
        _base_body,
        out_shape=(
            jax.ShapeDtypeStruct((N, D), jnp.float32),
            jax.ShapeDtypeStruct((N, D), jnp.float32),
        ),
    )(agg, deg_in, deg_out, w, b)


_RB = 2000
_NRB = N // _RB


def _final_body(base_ref, r_ref, hsn_ref, din_ref, k_ref, kcw_ref,
                w1a_ref, b1a_ref, w1b_ref, b1b_ref,
                w2a_ref, b2a_ref, w2b_ref, b2b_ref,
                fh_ref, z1_ref, z2_ref, dsum_ref):
    base = base_ref[...]
    agg2 = r_ref[...] + hsn_ref[...]
    lap = base - agg2 * lax.rsqrt(din_ref[...] + 1.0)
    k = k_ref[0, 0]
    kcw = kcw_ref[...]
    fh_ref[...] = base - (0.5 * (k + kcw)) * lap
    h1 = base - k * lap
    h2 = base - kcw * lap
    z1 = jnp.dot(jnp.maximum(jnp.dot(h1, w1a_ref[...],
                                     preferred_element_type=jnp.float32)
                             + b1a_ref[...], 0.0),
                 w1b_ref[...], preferred_element_type=jnp.float32) + b1b_ref[...]
    z2 = jnp.dot(jnp.maximum(jnp.dot(h2, w2a_ref[...],
                                     preferred_element_type=jnp.float32)
                             + b2a_ref[...], 0.0),
                 w2b_ref[...], preferred_element_type=jnp.float32) + b2b_ref[...]
    n1 = jnp.maximum(jnp.sqrt(jnp.sum(z1 * z1, axis=1, keepdims=True)), 1e-12)
    n2 = jnp.maximum(jnp.sqrt(jnp.sum(z2 * z2, axis=1, keepdims=True)), 1e-12)
    z1n = z1 / n1
    z2n = z2 / n2
    z1_ref[...] = z1n.astype(jnp.bfloat16)
    z2_ref[...] = z2n.astype(jnp.bfloat16)
    i = pl.program_id(0)

    @pl.when(i == 0)
    def _():
        dsum_ref[0, 0] = 0.0

    dsum_ref[0, 0] += jnp.sum(z1n * z2n)


def _tc_final(base, r, hsn, deg_in, k1, kcw, w1a, b1a, w1b, b1b,
              w2a, b2a, w2b, b2b):
    rb = lambda i: (i, 0)
    whole = lambda i: (0, 0)
    return pl.pallas_call(
        _final_body,
        grid=(_NRB,),
        in_specs=[
            pl.BlockSpec((_RB, D), rb),
            pl.BlockSpec((_RB, D), rb),
            pl.BlockSpec((_RB, D), rb),
            pl.BlockSpec((_RB, 1), rb),
            pl.BlockSpec(memory_space=pltpu.SMEM),
            pl.BlockSpec((1, D), whole),
            pl.BlockSpec((D, D), whole),
            pl.BlockSpec((1, D), whole),
            pl.BlockSpec((D, P), whole),
            pl.BlockSpec((1, P), whole),
            pl.BlockSpec((D, D), whole),
            pl.BlockSpec((1, D), whole),
            pl.BlockSpec((D, P), whole),
            pl.BlockSpec((1, P), whole),
        ],
        out_specs=(
            pl.BlockSpec((_RB, D), rb),
            pl.BlockSpec((_RB, P), rb),
            pl.BlockSpec((_RB, P), rb),
            pl.BlockSpec(memory_space=pltpu.SMEM),
        ),
        out_shape=(
            jax.ShapeDtypeStruct((N, D), jnp.float32),
            jax.ShapeDtypeStruct((N, P), jnp.bfloat16),
            jax.ShapeDtypeStruct((N, P), jnp.bfloat16),
            jax.ShapeDtypeStruct((1, 1), jnp.float32),
        ),
    )(base, r, hsn, deg_in, k1, kcw, w1a, b1a, w1b, b1b, w2a, b2a, w2b, b2b)


_LB = 400
_NLB = N // _LB


def _loss_body(z1_ref, z2_ref, dsum_ref, out_ref, colsum, racc):
    i = pl.program_id(0)
    s = lax.dot_general(z1_ref[...], z2_ref[...],
                        (((1,), (1,)), ((), ())),
                        preferred_element_type=jnp.float32) * (1.0 / TEMP)
    ex = jnp.exp(s)

    @pl.when(i == 0)
    def _():
        colsum[...] = jnp.zeros_like(colsum)
        racc[0] = 0.0

    colsum[...] += lax.dot_general(jnp.ones((1, _LB), jnp.float32), ex,
                                   (((1,), (0,)), ((), ())),
                                   preferred_element_type=jnp.float32)
    rs = lax.dot_general(ex, jnp.ones((N, 1), jnp.float32),
                         (((1,), (0,)), ((), ())),
                         preferred_element_type=jnp.float32)
    racc[0] += jnp.sum(jnp.log(rs))

    @pl.when(i == _NLB - 1)
    def _():
        closum = jnp.sum(jnp.log(colsum[...]))
        out_ref[0, 0] = (0.5 * (racc[0] + closum) / N
                         - dsum_ref[0, 0] * (1.0 / TEMP) / N)


def _tc_loss(z1n, z2n, dsum):
    return pl.pallas_call(
        _loss_body,
        grid=(_NLB,),
        in_specs=[
            pl.BlockSpec((_LB, P), lambda i: (i, 0)),
            pl.BlockSpec((N, P), lambda i: (0, 0)),
            pl.BlockSpec(memory_space=pltpu.SMEM),
        ],
        out_specs=pl.BlockSpec(memory_space=pltpu.SMEM),
        out_shape=jax.ShapeDtypeStruct((1, 1), jnp.float32),
        scratch_shapes=[
            pltpu.VMEM((1, N), jnp.float32),
            pltpu.SMEM((1,), jnp.float32),
        ],
    )(z1n, z2n, dsum)


def _interleave(h):
    hp = jnp.concatenate([h, jnp.zeros((1, D), jnp.float32)], axis=0)
    return hp.reshape((N + 1) * 2, P)


def kernel(x, edge_index, W_gnn, b_gnn, k_cross, K_cw,
           W1a, b1a, W1b, b1b, W2a, b2a, W2b, b2b):
    src = edge_index[0]
    dst = edge_index[1]
    pad = jnp.full((EPAD - E,), N, jnp.int32)
    srcp = jnp.concatenate([src, pad]).reshape(NS, CPT, CH)
    dstp = jnp.concatenate([dst, pad]).reshape(NS, CPT, CH)
    idx2 = jnp.stack([srcp, dstp])

    ones8 = jnp.ones((CH, P), jnp.float32)

    zrow = jnp.zeros((RPT, P), jnp.float32)

    degs = _sc_degrees(idx2, ones8, zrow)
    deg_out = degs[0, :N, 0:1]
    deg_in = degs[1, :N, 0:1]

    hn = _tc_hn(x, deg_out)
    agg2c = _sc_scatter(_interleave(hn), srcp, dstp, zrow)
    agg = jnp.concatenate([agg2c[0, :N], agg2c[1, :N]], axis=1)

    base_h, hsn = _tc_base(agg, deg_in, deg_out, W_gnn, b_gnn.reshape(1, D))

    r2c = _sc_scatter(_interleave(hsn), srcp, dstp, zrow)
    r = jnp.concatenate([r2c[0, :N], r2c[1, :N]], axis=1)

    final_h, z1n, z2n, dsum = _tc_final(
        base_h, r, hsn, deg_in, k_cross.reshape(1, 1), K_cw,
        W1a, b1a.reshape(1, D), W1b, b1b.reshape(1, P),
        W2a, b2a.reshape(1, D), W2b, b2b.reshape(1, P))

    loss = _tc_loss(z1n, z2n, dsum)[0, 0]
    return final_h, loss


# speedup vs baseline: 1.1507x; 1.1507x over previous
import functools

import jax
import jax.numpy as jnp
from jax import lax
from jax.experimental import pallas as pl
from jax.experimental.pallas import tpu as pltpu
from jax.experimental.pallas import tpu_sc as plsc

N = 10000
E = 160000
D = 256
P = 128
TEMP = 0.1

NC = 2
NS = 16
CH = 128
CPT = 80
EPAD = NS * CH * CPT
RPT = 632
NPAD = NS * RPT


def _deg_body(idx2_hbm, ones_hbm, zero8_hbm, out_hbm, idxv, onesv, hist):
    c = lax.axis_index("c")
    s = lax.axis_index("s")
    pltpu.sync_copy(ones_hbm, onesv)
    pltpu.sync_copy(idx2_hbm.at[c].at[s], idxv)
    pltpu.sync_copy(zero8_hbm, hist.at[pl.ds(s * RPT, RPT)])
    plsc.subcore_barrier()

    def body(j, carry):
        pltpu.sync_copy(onesv, hist.at[idxv.at[j]], add=True)
        return carry

    lax.fori_loop(0, CPT, body, 0)
    plsc.subcore_barrier()
    pltpu.sync_copy(hist.at[pl.ds(s * RPT, RPT)],
                    out_hbm.at[c].at[pl.ds(s * RPT, RPT)])


_HCPT = CPT // 2


def _scatter_body(rows_hbm, src_hbm, dst_hbm, zrow_hbm, out_hbm,
                  gidx_all, didx_all, rows0, rows1, acc,
                  gsem0, gsem1, ssem0, ssem1):
    c = lax.axis_index("c")
    s = lax.axis_index("s")
    rows_bufs = (rows0, rows1)
    gsems = (gsem0, gsem1)
    ssems = (ssem0, ssem1)

    def prep_half(h):
        pltpu.sync_copy(src_hbm.at[s].at[pl.ds(h * _HCPT, _HCPT)], gidx_all)
        pltpu.sync_copy(dst_hbm.at[s].at[pl.ds(h * _HCPT, _HCPT)], didx_all)

        def xform(r, carry):
            for i in range(CH // 16):
                v = gidx_all[r, pl.ds(i * 16, 16)]
                gidx_all[r, pl.ds(i * 16, 16)] = v * 2 + c
            return carry

        lax.fori_loop(0, _HCPT, xform, 0)

    def prime(h):
        pltpu.async_copy(rows_hbm.at[gidx_all.at[0]], rows0, gsems[0])
        pltpu.async_copy(rows_hbm.at[gidx_all.at[1]], rows1, gsems[1])

    def run_half():
        def body(g, carry):
            for b in range(2):
                j = 2 * g + b
                pltpu.make_async_copy(rows_hbm.at[gidx_all.at[0]],
                                      rows_bufs[b], gsems[b]).wait()
                pltpu.sync_copy(rows_bufs[b], acc.at[didx_all.at[j]],
                                add=True)

                @pl.when(j + 2 < _HCPT)
                def _():
                    pltpu.async_copy(rows_hbm.at[gidx_all.at[j + 2]],
                                     rows_bufs[b], gsems[b])
            return carry

        lax.fori_loop(0, _HCPT // 2, body, 0)

    pltpu.sync_copy(zrow_hbm, acc.at[pl.ds(s * RPT, RPT)])
    prep_half(0)
    prime(0)
    plsc.subcore_barrier()
    run_half()
    prep_half(1)
    prime(1)
    run_half()
    plsc.subcore_barrier()
    pltpu.sync_copy(acc.at[pl.ds(s * RPT, RPT)],
                    out_hbm.at[c].at[pl.ds(s * RPT, RPT)])


@functools.cache
def _sc_kernels():
    mesh = plsc.VectorSubcoreMesh(core_axis_name="c", subcore_axis_name="s",
                                  num_cores=NC, num_subcores=NS)
    deg = pl.kernel(
        _deg_body,
        out_type=jax.ShapeDtypeStruct((NC, NPAD, P), jnp.float32),
        mesh=mesh,
        scratch_types=[
            pltpu.VMEM((CPT, CH), jnp.int32),
            pltpu.VMEM((CH, P), jnp.float32),
            pltpu.VMEM_SHARED((NPAD, P), jnp.float32),
        ],
    )
    scat = pl.kernel(
        _scatter_body,
        out_type=jax.ShapeDtypeStruct((NC, NPAD, P), jnp.float32),
        mesh=mesh,
        scratch_types=[
            pltpu.VMEM((_HCPT, CH), jnp.int32),
            pltpu.VMEM((_HCPT, CH), jnp.int32),
            pltpu.VMEM((CH, P), jnp.float32),
            pltpu.VMEM((CH, P), jnp.float32),
            pltpu.VMEM_SHARED((NPAD, P), jnp.float32),
            pltpu.SemaphoreType.DMA,
            pltpu.SemaphoreType.DMA,
            pltpu.SemaphoreType.DMA,
            pltpu.SemaphoreType.DMA,
        ],
    )
    return deg, scat


def _sc_degrees(idx2, ones8, zrow):
    return _sc_kernels()[0](idx2, ones8, zrow)


def _sc_scatter(rows_il, srcp, dstp, zrow):
    return _sc_kernels()[1](rows_il, srcp, dstp, zrow)


def _hn_body(x_ref, dout_ref, hn_ref):
    sc = lax.rsqrt(jnp.maximum(dout_ref[...], 1.0))
    hn_ref[...] = x_ref[...] * sc


def _tc_hn(x, deg_out):
    return pl.pallas_call(
        _hn_body,
        out_shape=jax.ShapeDtypeStruct((N, D), jnp.float32),
    )(x, deg_out)


def _base_body(agg_ref, din_ref, dout_ref, w_ref, b_ref, base_ref, hsn_ref):
    s_in = lax.rsqrt(jnp.maximum(din_ref[...], 1.0))
    scaled = agg_ref[...] * s_in
    base = jnp.dot(scaled, w_ref[...],
                   preferred_element_type=jnp.float32) + b_ref[...]
    base_ref[...] = base
    hsn_ref[...] = base * lax.rsqrt(dout_ref[...] + 1.0)


def _tc_base(agg, deg_in, deg_out, w, b):
    return pl.pallas_call(
        _base_body,
        out_shape=(
            jax.ShapeDtypeStruct((N, D), jnp.float32),
            jax.ShapeDtypeStruct((N, D), jnp.float32),
        ),
    )(agg, deg_in, deg_out, w, b)


_RB = 2000
_NRB = N // _RB


def _final_body(base_ref, r_ref, hsn_ref, din_ref, k_ref, kcw_ref,
                w1a_ref, b1a_ref, w1b_ref, b1b_ref,
                w2a_ref, b2a_ref, w2b_ref, b2b_ref,
                fh_ref, z1_ref, z2_ref, dsum_ref):
    base = base_ref[...]
    agg2 = r_ref[...] + hsn_ref[...]
    lap = base - agg2 * lax.rsqrt(din_ref[...] + 1.0)
    k = k_ref[0, 0]
    kcw = kcw_ref[...]
    fh_ref[...] = base - (0.5 * (k + kcw)) * lap
    h1 = base - k * lap
    h2 = base - kcw * lap
    z1 = jnp.dot(jnp.maximum(jnp.dot(h1, w1a_ref[...],
                                     preferred_element_type=jnp.float32)
                             + b1a_ref[...], 0.0),
                 w1b_ref[...], preferred_element_type=jnp.float32) + b1b_ref[...]
    z2 = jnp.dot(jnp.maximum(jnp.dot(h2, w2a_ref[...],
                                     preferred_element_type=jnp.float32)
                             + b2a_ref[...], 0.0),
                 w2b_ref[...], preferred_element_type=jnp.float32) + b2b_ref[...]
    n1 = jnp.maximum(jnp.sqrt(jnp.sum(z1 * z1, axis=1, keepdims=True)), 1e-12)
    n2 = jnp.maximum(jnp.sqrt(jnp.sum(z2 * z2, axis=1, keepdims=True)), 1e-12)
    z1n = z1 / n1
    z2n = z2 / n2
    z1_ref[...] = z1n.astype(jnp.bfloat16)
    z2_ref[...] = z2n.astype(jnp.bfloat16)
    i = pl.program_id(0)

    @pl.when(i == 0)
    def _():
        dsum_ref[0, 0] = 0.0

    dsum_ref[0, 0] += jnp.sum(z1n * z2n)


def _tc_final(base, r, hsn, deg_in, k1, kcw, w1a, b1a, w1b, b1b,
              w2a, b2a, w2b, b2b):
    rb = lambda i: (i, 0)
    whole = lambda i: (0, 0)
    return pl.pallas_call(
        _final_body,
        grid=(_NRB,),
        in_specs=[
            pl.BlockSpec((_RB, D), rb),
            pl.BlockSpec((_RB, D), rb),
            pl.BlockSpec((_RB, D), rb),
            pl.BlockSpec((_RB, 1), rb),
            pl.BlockSpec(memory_space=pltpu.SMEM),
            pl.BlockSpec((1, D), whole),
            pl.BlockSpec((D, D), whole),
            pl.BlockSpec((1, D), whole),
            pl.BlockSpec((D, P), whole),
            pl.BlockSpec((1, P), whole),
            pl.BlockSpec((D, D), whole),
            pl.BlockSpec((1, D), whole),
            pl.BlockSpec((D, P), whole),
            pl.BlockSpec((1, P), whole),
        ],
        out_specs=(
            pl.BlockSpec((_RB, D), rb),
            pl.BlockSpec((_RB, P), rb),
            pl.BlockSpec((_RB, P), rb),
            pl.BlockSpec(memory_space=pltpu.SMEM),
        ),
        out_shape=(
            jax.ShapeDtypeStruct((N, D), jnp.float32),
            jax.ShapeDtypeStruct((N, P), jnp.bfloat16),
            jax.ShapeDtypeStruct((N, P), jnp.bfloat16),
            jax.ShapeDtypeStruct((1, 1), jnp.float32),
        ),
    )(base, r, hsn, deg_in, k1, kcw, w1a, b1a, w1b, b1b, w2a, b2a, w2b, b2b)


_LB = 400
_NLB = N // _LB


def _loss_body(z1_ref, z2_ref, dsum_ref, out_ref, colsum, racc):
    i = pl.program_id(0)
    s = lax.dot_general(z1_ref[...], z2_ref[...],
                        (((1,), (1,)), ((), ())),
                        preferred_element_type=jnp.float32) * (1.0 / TEMP)
    ex = jnp.exp(s)

    @pl.when(i == 0)
    def _():
        colsum[...] = jnp.zeros_like(colsum)
        racc[0] = 0.0

    colsum[...] += jnp.sum(ex, axis=0, keepdims=True)
    rs = jnp.sum(ex, axis=1)
    racc[0] += jnp.sum(jnp.log(rs))

    @pl.when(i == _NLB - 1)
    def _():
        closum = jnp.sum(jnp.log(colsum[...]))
        out_ref[0, 0] = (0.5 * (racc[0] + closum) / N
                         - dsum_ref[0, 0] * (1.0 / TEMP) / N)


def _tc_loss(z1n, z2n, dsum):
    return pl.pallas_call(
        _loss_body,
        grid=(_NLB,),
        in_specs=[
            pl.BlockSpec((_LB, P), lambda i: (i, 0)),
            pl.BlockSpec((N, P), lambda i: (0, 0)),
            pl.BlockSpec(memory_space=pltpu.SMEM),
        ],
        out_specs=pl.BlockSpec(memory_space=pltpu.SMEM),
        out_shape=jax.ShapeDtypeStruct((1, 1), jnp.float32),
        scratch_shapes=[
            pltpu.VMEM((1, N), jnp.float32),
            pltpu.SMEM((1,), jnp.float32),
        ],
    )(z1n, z2n, dsum)


def _interleave(h):
    hp = jnp.concatenate([h, jnp.zeros((1, D), jnp.float32)], axis=0)
    return hp.reshape((N + 1) * 2, P)


def kernel(x, edge_index, W_gnn, b_gnn, k_cross, K_cw,
           W1a, b1a, W1b, b1b, W2a, b2a, W2b, b2b):
    src = edge_index[0]
    dst = edge_index[1]
    pad = jnp.full((EPAD - E,), N, jnp.int32)
    srcp = jnp.concatenate([src, pad]).reshape(NS, CPT, CH)
    dstp = jnp.concatenate([dst, pad]).reshape(NS, CPT, CH)
    idx2 = jnp.stack([srcp, dstp])

    ones8 = jnp.ones((CH, P), jnp.float32)

    zrow = jnp.zeros((RPT, P), jnp.float32)

    degs = _sc_degrees(idx2, ones8, zrow)
    deg_out = degs[0, :N, 0:1]
    deg_in = degs[1, :N, 0:1]

    hn = _tc_hn(x, deg_out)
    agg2c = _sc_scatter(_interleave(hn), srcp, dstp, zrow)
    agg = jnp.concatenate([agg2c[0, :N], agg2c[1, :N]], axis=1)

    base_h, hsn = _tc_base(agg, deg_in, deg_out, W_gnn, b_gnn.reshape(1, D))

    r2c = _sc_scatter(_interleave(hsn), srcp, dstp, zrow)
    r = jnp.concatenate([r2c[0, :N], r2c[1, :N]], axis=1)

    final_h, z1n, z2n, dsum = _tc_final(
        base_h, r, hsn, deg_in, k_cross.reshape(1, 1), K_cw,
        W1a, b1a.reshape(1, D), W1b, b1b.reshape(1, P),
        W2a, b2a.reshape(1, D), W2b, b2b.reshape(1, P))

    loss = _tc_loss(z1n, z2n, dsum)[0, 0]
    return final_h, loss


# speedup vs baseline: 1.2461x; 1.0829x over previous
import functools

import jax
import jax.numpy as jnp
from jax import lax
from jax.experimental import pallas as pl
from jax.experimental.pallas import tpu as pltpu
from jax.experimental.pallas import tpu_sc as plsc

N = 10000
E = 160000
D = 256
P = 128
TEMP = 0.1

NC = 2
NS = 16
CH = 128
CPT = 80
EPAD = NS * CH * CPT
RPT = 632
NPAD = NS * RPT
NPR = N + 8


def _deg_body(idx2_hbm, ones_hbm, zero8_hbm, out_hbm, idxv, onesv, hist):
    c = lax.axis_index("c")
    s = lax.axis_index("s")
    pltpu.sync_copy(ones_hbm, onesv)
    pltpu.sync_copy(idx2_hbm.at[c].at[s], idxv)
    pltpu.sync_copy(zero8_hbm, hist.at[pl.ds(s * RPT, RPT)])
    plsc.subcore_barrier()

    def body(j, carry):
        pltpu.sync_copy(onesv, hist.at[idxv.at[j]], add=True)
        return carry

    lax.fori_loop(0, CPT, body, 0)
    plsc.subcore_barrier()
    pltpu.sync_copy(hist.at[pl.ds(s * RPT, RPT)],
                    out_hbm.at[c].at[pl.ds(s * RPT, RPT)])


_HCPT = CPT // 2


def _scatter_body(rows_hbm, src_hbm, dst_hbm, zrow_hbm, out_hbm,
                  gidx_all, didx_all, rows0, rows1, acc,
                  gsem0, gsem1, ssem0, ssem1):
    c = lax.axis_index("c")
    s = lax.axis_index("s")
    rows_bufs = (rows0, rows1)
    gsems = (gsem0, gsem1)
    ssems = (ssem0, ssem1)

    def prep_half(h):
        pltpu.sync_copy(src_hbm.at[s].at[pl.ds(h * _HCPT, _HCPT)], gidx_all)
        pltpu.sync_copy(dst_hbm.at[s].at[pl.ds(h * _HCPT, _HCPT)], didx_all)

        def xform(r, carry):
            for i in range(CH // 16):
                v = gidx_all[r, pl.ds(i * 16, 16)]
                gidx_all[r, pl.ds(i * 16, 16)] = v * 2 + c
            return carry

        lax.fori_loop(0, _HCPT, xform, 0)

    def prime(h):
        pltpu.async_copy(rows_hbm.at[gidx_all.at[0]], rows0, gsems[0])
        pltpu.async_copy(rows_hbm.at[gidx_all.at[1]], rows1, gsems[1])

    def run_half():
        def body(g, carry):
            for b in range(2):
                j = 2 * g + b
                pltpu.make_async_copy(rows_hbm.at[gidx_all.at[0]],
                                      rows_bufs[b], gsems[b]).wait()
                pltpu.sync_copy(rows_bufs[b], acc.at[didx_all.at[j]],
                                add=True)

                @pl.when(j + 2 < _HCPT)
                def _():
                    pltpu.async_copy(rows_hbm.at[gidx_all.at[j + 2]],
                                     rows_bufs[b], gsems[b])
            return carry

        lax.fori_loop(0, _HCPT // 2, body, 0)

    pltpu.sync_copy(zrow_hbm, acc.at[pl.ds(s * RPT, RPT)])
    prep_half(0)
    prime(0)
    plsc.subcore_barrier()
    run_half()
    prep_half(1)
    prime(1)
    run_half()
    plsc.subcore_barrier()
    pltpu.sync_copy(acc.at[pl.ds(s * RPT, RPT)],
                    out_hbm.at[c].at[pl.ds(s * RPT, RPT)])


@functools.cache
def _sc_kernels():
    mesh = plsc.VectorSubcoreMesh(core_axis_name="c", subcore_axis_name="s",
                                  num_cores=NC, num_subcores=NS)
    deg = pl.kernel(
        _deg_body,
        out_type=jax.ShapeDtypeStruct((NC, NPAD, P), jnp.float32),
        mesh=mesh,
        scratch_types=[
            pltpu.VMEM((CPT, CH), jnp.int32),
            pltpu.VMEM((CH, P), jnp.float32),
            pltpu.VMEM_SHARED((NPAD, P), jnp.float32),
        ],
    )
    scat = pl.kernel(
        _scatter_body,
        out_type=jax.ShapeDtypeStruct((NC, NPAD, P), jnp.float32),
        mesh=mesh,
        scratch_types=[
            pltpu.VMEM((_HCPT, CH), jnp.int32),
            pltpu.VMEM((_HCPT, CH), jnp.int32),
            pltpu.VMEM((CH, P), jnp.float32),
            pltpu.VMEM((CH, P), jnp.float32),
            pltpu.VMEM_SHARED((NPAD, P), jnp.float32),
            pltpu.SemaphoreType.DMA,
            pltpu.SemaphoreType.DMA,
            pltpu.SemaphoreType.DMA,
            pltpu.SemaphoreType.DMA,
        ],
    )
    return deg, scat


def _sc_degrees(idx2, ones8, zrow):
    return _sc_kernels()[0](idx2, ones8, zrow)


def _sc_scatter(rows_il, srcp, dstp, zrow):
    return _sc_kernels()[1](rows_il, srcp, dstp, zrow)


def _hn_body(x_ref, dout_ref, hn_ref):
    sc = lax.rsqrt(jnp.maximum(dout_ref[...], 1.0))
    hn_ref[...] = jnp.concatenate(
        [x_ref[...] * sc, jnp.zeros((NPR - N, D), jnp.float32)], axis=0)


def _tc_hn(x, deg_out):
    return pl.pallas_call(
        _hn_body,
        out_shape=jax.ShapeDtypeStruct((NPR, D), jnp.float32),
    )(x, deg_out)


def _base_body(agg_ref, din_ref, dout_ref, w_ref, b_ref, base_ref, hsn_ref):
    s_in = lax.rsqrt(jnp.maximum(din_ref[...], 1.0))
    agg = jnp.concatenate([agg_ref[0, :N, :], agg_ref[1, :N, :]], axis=1)
    base = jnp.dot(agg * s_in, w_ref[...],
                   preferred_element_type=jnp.float32) + b_ref[...]
    base_ref[...] = base
    hsn = base * lax.rsqrt(dout_ref[...] + 1.0)
    hsn_ref[...] = jnp.concatenate(
        [hsn, jnp.zeros((NPR - N, D), jnp.float32)], axis=0)


def _tc_base(agg2c, deg_in, deg_out, w, b):
    return pl.pallas_call(
        _base_body,
        out_shape=(
            jax.ShapeDtypeStruct((N, D), jnp.float32),
            jax.ShapeDtypeStruct((NPR, D), jnp.float32),
        ),
    )(agg2c, deg_in, deg_out, w, b)


_RB = 2000
_NRB = N // _RB


def _final_body(base_ref, r_ref, hsn_ref, din_ref, k_ref, kcw_ref,
                w1a_ref, b1a_ref, w1b_ref, b1b_ref,
                w2a_ref, b2a_ref, w2b_ref, b2b_ref,
                fh_ref, z1_ref, z2_ref, dsum_ref):
    base = base_ref[...]
    r = jnp.concatenate([r_ref[0], r_ref[1]], axis=1)
    agg2 = r + hsn_ref[...]
    lap = base - agg2 * lax.rsqrt(din_ref[...] + 1.0)
    k = k_ref[0, 0]
    kcw = kcw_ref[...]
    fh_ref[...] = base - (0.5 * (k + kcw)) * lap
    h1 = base - k * lap
    h2 = base - kcw * lap
    z1 = jnp.dot(jnp.maximum(jnp.dot(h1, w1a_ref[...],
                                     preferred_element_type=jnp.float32)
                             + b1a_ref[...], 0.0),
                 w1b_ref[...], preferred_element_type=jnp.float32) + b1b_ref[...]
    z2 = jnp.dot(jnp.maximum(jnp.dot(h2, w2a_ref[...],
                                     preferred_element_type=jnp.float32)
                             + b2a_ref[...], 0.0),
                 w2b_ref[...], preferred_element_type=jnp.float32) + b2b_ref[...]
    n1 = jnp.maximum(jnp.sqrt(jnp.sum(z1 * z1, axis=1, keepdims=True)), 1e-12)
    n2 = jnp.maximum(jnp.sqrt(jnp.sum(z2 * z2, axis=1, keepdims=True)), 1e-12)
    z1n = z1 / n1
    z2n = z2 / n2
    z1_ref[...] = z1n.astype(jnp.bfloat16)
    z2_ref[...] = z2n.astype(jnp.bfloat16)
    i = pl.program_id(0)

    @pl.when(i == 0)
    def _():
        dsum_ref[0, 0] = 0.0

    dsum_ref[0, 0] += jnp.sum(z1n * z2n)


def _tc_final(base, r, hsn, deg_in, k1, kcw, w1a, b1a, w1b, b1b,
              w2a, b2a, w2b, b2b):
    rb = lambda i: (i, 0)
    whole = lambda i: (0, 0)
    return pl.pallas_call(
        _final_body,
        grid=(_NRB,),
        in_specs=[
            pl.BlockSpec((_RB, D), rb),
            pl.BlockSpec((2, _RB, P), lambda i: (0, i, 0)),
            pl.BlockSpec((_RB, D), rb),
            pl.BlockSpec((_RB, 1), rb),
            pl.BlockSpec(memory_space=pltpu.SMEM),
            pl.BlockSpec((1, D), whole),
            pl.BlockSpec((D, D), whole),
            pl.BlockSpec((1, D), whole),
            pl.BlockSpec((D, P), whole),
            pl.BlockSpec((1, P), whole),
            pl.BlockSpec((D, D), whole),
            pl.BlockSpec((1, D), whole),
            pl.BlockSpec((D, P), whole),
            pl.BlockSpec((1, P), whole),
        ],
        out_specs=(
            pl.BlockSpec((_RB, D), rb),
            pl.BlockSpec((_RB, P), rb),
            pl.BlockSpec((_RB, P), rb),
            pl.BlockSpec(memory_space=pltpu.SMEM),
        ),
        out_shape=(
            jax.ShapeDtypeStruct((N, D), jnp.float32),
            jax.ShapeDtypeStruct((N, P), jnp.bfloat16),
            jax.ShapeDtypeStruct((N, P), jnp.bfloat16),
            jax.ShapeDtypeStruct((1, 1), jnp.float32),
        ),
    )(base, r, hsn, deg_in, k1, kcw, w1a, b1a, w1b, b1b, w2a, b2a, w2b, b2b)


_LB = 400
_NLB = N // _LB


def _loss_body(z1_ref, z2_ref, dsum_ref, out_ref, colsum, racc):
    i = pl.program_id(0)
    s = lax.dot_general(z1_ref[...], z2_ref[...],
                        (((1,), (1,)), ((), ())),
                        preferred_element_type=jnp.float32) * (1.0 / TEMP)
    ex = jnp.exp(s)

    @pl.when(i == 0)
    def _():
        colsum[...] = jnp.zeros_like(colsum)
        racc[0] = 0.0

    colsum[...] += jnp.sum(ex, axis=0, keepdims=True)
    rs = jnp.sum(ex, axis=1)
    racc[0] += jnp.sum(jnp.log(rs))

    @pl.when(i == _NLB - 1)
    def _():
        closum = jnp.sum(jnp.log(colsum[...]))
        out_ref[0, 0] = (0.5 * (racc[0] + closum) / N
                         - dsum_ref[0, 0] * (1.0 / TEMP) / N)


def _tc_loss(z1n, z2n, dsum):
    return pl.pallas_call(
        _loss_body,
        grid=(_NLB,),
        in_specs=[
            pl.BlockSpec((_LB, P), lambda i: (i, 0)),
            pl.BlockSpec((N, P), lambda i: (0, 0)),
            pl.BlockSpec(memory_space=pltpu.SMEM),
        ],
        out_specs=pl.BlockSpec(memory_space=pltpu.SMEM),
        out_shape=jax.ShapeDtypeStruct((1, 1), jnp.float32),
        scratch_shapes=[
            pltpu.VMEM((1, N), jnp.float32),
            pltpu.SMEM((1,), jnp.float32),
        ],
    )(z1n, z2n, dsum)


def _interleave(h):
    return h.reshape(NPR * 2, P)


def kernel(x, edge_index, W_gnn, b_gnn, k_cross, K_cw,
           W1a, b1a, W1b, b1b, W2a, b2a, W2b, b2b):
    src = edge_index[0]
    dst = edge_index[1]
    pad = jnp.full((EPAD - E,), N, jnp.int32)
    srcp = jnp.concatenate([src, pad]).reshape(NS, CPT, CH)
    dstp = jnp.concatenate([dst, pad]).reshape(NS, CPT, CH)
    idx2 = jnp.stack([srcp, dstp])

    ones8 = jnp.ones((CH, P), jnp.float32)

    zrow = jnp.zeros((RPT, P), jnp.float32)

    degs = _sc_degrees(idx2, ones8, zrow)
    deg_out = degs[0, :N, 0:1]
    deg_in = degs[1, :N, 0:1]

    hn = _tc_hn(x, deg_out)
    agg2c = _sc_scatter(_interleave(hn), srcp, dstp, zrow)

    base_h, hsn = _tc_base(agg2c, deg_in, deg_out, W_gnn, b_gnn.reshape(1, D))

    r2c = _sc_scatter(_interleave(hsn), srcp, dstp, zrow)

    final_h, z1n, z2n, dsum = _tc_final(
        base_h, r2c, hsn[:N], deg_in, k_cross.reshape(1, 1), K_cw,
        W1a, b1a.reshape(1, D), W1b, b1b.reshape(1, P),
        W2a, b2a.reshape(1, D), W2b, b2b.reshape(1, P))

    loss = _tc_loss(z1n, z2n, dsum)[0, 0]
    return final_h, loss
